# baseline (speedup 1.0000x reference)
import functools

import jax
import jax.numpy as jnp
from jax import lax
from jax.experimental import pallas as pl
from jax.experimental.pallas import tpu as pltpu
from jax.experimental.pallas import tpu_sc as plsc

_N = 4096
_K = 1024
_T = 4.0
_L = 16
_NV = _N // _L


def _sc_probe_body(alpha_hbm, out_hbm, w_v, m_v):
    cid = lax.axis_index("c")
    sid = lax.axis_index("s")

    @pl.when(jnp.logical_and(cid == 0, sid == 0))
    def _():
        pltpu.sync_copy(alpha_hbm, w_v)
        pltpu.sync_copy(w_v, out_hbm)


def _sc_mask_body(alpha_hbm, out_hbm, w_v, m_v):
    cid = lax.axis_index("c")
    sid = lax.axis_index("s")

    @pl.when(jnp.logical_and(cid == 0, sid == 0))
    def _():
        pltpu.sync_copy(alpha_hbm, w_v)

        def p1(i, mx):
            u = w_v[pl.ds(i * _L, _L)] * (1.0 / _T)
            w_v[pl.ds(i * _L, _L)] = u
            return jnp.maximum(mx, u)

        mxv = lax.fori_loop(0, _NV, p1, jnp.full((_L,), -jnp.inf, jnp.float32))
        mx = jnp.max(mxv)

        def p2(i, sv):
            e = jnp.exp(w_v[pl.ds(i * _L, _L)] - mx)
            w_v[pl.ds(i * _L, _L)] = e
            return sv + e

        sv = lax.fori_loop(0, _NV, p2, jnp.zeros((_L,), jnp.float32))
        s = jnp.sum(sv)

        def p3(i, c):
            w_v[pl.ds(i * _L, _L)] = w_v[pl.ds(i * _L, _L)] / s
            return c

        lax.fori_loop(0, _NV, p3, jnp.int32(0))

        def wbits(i):
            return plsc.bitcast(w_v[pl.ds(i * _L, _L)], jnp.int32)

        def count_ge(v):
            def b(i, acc):
                return acc + (wbits(i) >= v).astype(jnp.int32)

            acc = lax.fori_loop(0, _NV, b, jnp.zeros((_L,), jnp.int32))
            return jnp.sum(acc)

        def bstep(_, lohi):
            lo, hi = lohi
            mid = lo + (hi - lo) // 2
            ok = count_ge(mid) >= _K
            return jnp.where(ok, mid, lo), jnp.where(ok, hi, mid)

        lo, _hi = lax.fori_loop(
            0, 30, bstep, (jnp.int32(0), jnp.int32(1 << 30)))
        tb = lo

        def cnt2(i, acc):
            a_ge, a_eq = acc
            b = wbits(i)
            return (a_ge + (b >= tb).astype(jnp.int32),
                    a_eq + (b == tb).astype(jnp.int32))

        a_ge, a_eq = lax.fori_loop(
            0, _NV, cnt2,
            (jnp.zeros((_L,), jnp.int32), jnp.zeros((_L,), jnp.int32)))
        n_ge = jnp.sum(a_ge)
        n_eq = jnp.sum(a_eq)
        need = _K - (n_ge - n_eq)

        def tp(j, after):
            r = _NV - 1 - j
            b = wbits(r)
            w = w_v[pl.ds(r * _L, _L)]
            tie = (b == tb).astype(jnp.int32)
            csum = jnp.cumsum(tie)
            tot = jnp.sum(tie)
            after_elem = after + (tot - csum)
            keep = jnp.logical_or(
                b > tb, jnp.logical_and(tie == 1, after_elem < need))
            m_v[pl.ds(r * _L, _L)] = jnp.where(keep, w, 0.0)
            return after + tot

        lax.fori_loop(0, _NV, tp, jnp.int32(0))
        pltpu.sync_copy(m_v, out_hbm)


_sc_mask = functools.partial(
    pl.kernel,
    mesh=plsc.VectorSubcoreMesh(core_axis_name="c", subcore_axis_name="s"),
    out_type=jax.ShapeDtypeStruct((_N,), jnp.float32),
    scratch_types=[
        pltpu.VMEM((_N,), jnp.float32),
        pltpu.VMEM((_N,), jnp.float32),
    ],
    compiler_params=pltpu.CompilerParams(needs_layout_passes=False),
)(_sc_mask_body)


_N_ROWS = 16384
_R_TC = 13312
_R_SC = _N_ROWS - _R_TC
_NW = 32
_RPT = _R_SC // _NW
_CH = 4
_NCH = _RPT // _CH


def _sc_stream_body(x_hbm, mask_hbm, bias_hbm, out_hbm,
                    mask_v, bias_v, xin0, xin1, ob0, ob1,
                    isem0, isem1, osem0, osem1):
    cid = lax.axis_index("c")
    sid = lax.axis_index("s")
    wid = sid * 2 + cid
    row0 = _R_TC + wid * _RPT

    pltpu.sync_copy(mask_hbm, mask_v)
    pltpu.sync_copy(bias_hbm, bias_v)

    def in_dma(g, buf, sem):
        return pltpu.make_async_copy(
            x_hbm.at[pl.ds(row0 + g * _CH, _CH)], buf, sem)

    def out_dma(g, buf, sem):
        return pltpu.make_async_copy(
            buf, out_hbm.at[pl.ds(wid * _RPT + g * _CH, _CH)], sem)

    def compute(xin, ob):
        def col(c, carry):
            m = mask_v[pl.ds(c * _L, _L)]
            b = bias_v[pl.ds(c * _L, _L)]
            for r in range(_CH):
                ob[r, pl.ds(c * _L, _L)] = xin[r, pl.ds(c * _L, _L)] * m + b
            return carry

        lax.fori_loop(0, _NV, col, jnp.int32(0))

    bufs = ((xin0, isem0, ob0, osem0), (xin1, isem1, ob1, osem1))

    in_dma(0, xin0, isem0).start()

    def chunk(gp, carry):
        for slot in range(2):
            xin, isem, ob, osem = bufs[slot]
            g = gp * 2 + slot
            nxin, nisem = bufs[1 - slot][0], bufs[1 - slot][1]

            @pl.when(g + 1 < _NCH)
            def _():
                in_dma(g + 1, nxin, nisem).start()

            in_dma(g, xin, isem).wait()

            @pl.when(g >= 2)
            def _():
                out_dma(g - 2, ob, osem).wait()

            compute(xin, ob)
            out_dma(g, ob, osem).start()
        return carry

    lax.fori_loop(0, _NCH // 2, chunk, jnp.int32(0))
    out_dma(_NCH - 2, ob0, osem0).wait()
    out_dma(_NCH - 1, ob1, osem1).wait()


_sc_stream = functools.partial(
    pl.kernel,
    mesh=plsc.VectorSubcoreMesh(core_axis_name="c", subcore_axis_name="s"),
    out_type=jax.ShapeDtypeStruct((_R_SC, _N), jnp.float32),
    scratch_types=[
        pltpu.VMEM((_N,), jnp.float32),
        pltpu.VMEM((_N,), jnp.float32),
        pltpu.VMEM((_CH, _N), jnp.float32),
        pltpu.VMEM((_CH, _N), jnp.float32),
        pltpu.VMEM((_CH, _N), jnp.float32),
        pltpu.VMEM((_CH, _N), jnp.float32),
        pltpu.SemaphoreType.DMA,
        pltpu.SemaphoreType.DMA,
        pltpu.SemaphoreType.DMA,
        pltpu.SemaphoreType.DMA,
    ],
    compiler_params=pltpu.CompilerParams(needs_layout_passes=False),
)(_sc_stream_body)


_BLK = 512


def _stream_body(x_ref, m_ref, b_ref, o_ref):
    o_ref[...] = x_ref[...] * m_ref[...] + b_ref[...]


def _tc_stream(x, mask, bias):
    nt = _R_TC
    return pl.pallas_call(
        _stream_body,
        grid=(nt // _BLK,),
        in_specs=[
            pl.BlockSpec((_BLK, _N), lambda i: (i, 0)),
            pl.BlockSpec((1, _N), lambda i: (0, 0)),
            pl.BlockSpec((1, _N), lambda i: (0, 0)),
        ],
        out_specs=pl.BlockSpec((_BLK, _N), lambda i: (i, 0)),
        out_shape=jax.ShapeDtypeStruct((nt, _N), jnp.float32),
        compiler_params=pltpu.CompilerParams(
            dimension_semantics=("parallel",)),
    )(x, mask, bias)


def kernel(x, alpha, bias):
    mask = _sc_mask(alpha.reshape(_N))
    z_sc = _sc_stream(x, mask, bias.reshape(_N))
    z_tc = _tc_stream(x, mask.reshape(1, _N), bias)
    return jnp.concatenate([z_tc, z_sc], axis=0)


# speedup vs baseline: 1.8633x; 1.8633x over previous
import functools

import jax
import jax.numpy as jnp
from jax import lax
from jax.experimental import pallas as pl
from jax.experimental.pallas import tpu as pltpu
from jax.experimental.pallas import tpu_sc as plsc

_N = 4096
_K = 1024
_T = 4.0
_L = 16
_NV = _N // _L


def _sc_mask_body(alpha_hbm, out_hbm, w_v, m_v, h_v):
    cid = lax.axis_index("c")
    sid = lax.axis_index("s")

    @pl.when(jnp.logical_and(cid == 0, sid == 0))
    def _():
        pltpu.sync_copy(alpha_hbm, w_v)
        iota16 = jnp.arange(_L, dtype=jnp.int32)
        lane_off = iota16 * 256
        ones = jnp.full((_L,), 1, jnp.int32)

        def z0(i, c):
            h_v[pl.ds(i * _L, _L)] = jnp.zeros((_L,), jnp.int32)
            return c

        lax.fori_loop(0, _NV, z0, jnp.int32(0), unroll=8)

        def p1(i, mm):
            mxv, mnv = mm
            u = w_v[pl.ds(i * _L, _L)] * (1.0 / _T)
            w_v[pl.ds(i * _L, _L)] = u
            return jnp.maximum(mxv, u), jnp.minimum(mnv, u)

        mxv, mnv = lax.fori_loop(
            0, _NV, p1,
            (jnp.full((_L,), -jnp.inf, jnp.float32),
             jnp.full((_L,), jnp.inf, jnp.float32)), unroll=8)
        mx = jnp.max(mxv)
        mn = jnp.min(mnv)

        def p2(i, sv):
            e = jnp.exp(w_v[pl.ds(i * _L, _L)] - mx)
            w_v[pl.ds(i * _L, _L)] = e
            return sv + e

        sv = lax.fori_loop(0, _NV, p2, jnp.zeros((_L,), jnp.float32),
                           unroll=8)
        s = jnp.sum(sv)

        lob = jnp.max(plsc.bitcast(jnp.exp(jnp.full((_L,), mn) - mx),
                                   jnp.int32))
        hib = jnp.max(plsc.bitcast(jnp.exp(jnp.zeros((_L,), jnp.float32)),
                                   jnp.int32)) + 1

        def ebits(i):
            return plsc.bitcast(w_v[pl.ds(i * _L, _L)], jnp.int32)

        def log2_shift(rng):
            rf = (jnp.zeros((_L,), jnp.int32) + rng).astype(jnp.float32)
            rexp = ((plsc.bitcast(rf, jnp.int32) >> 23) & 255) - 127
            return jnp.max(jnp.maximum(rexp - 7, 0))

        Kp = jnp.int32(_K)
        for _lvl in range(5):
            sh = log2_shift(hib - lob - 1)

            def build(i, c, sh=sh, lob=lob, hib=hib):
                b = ebits(i)
                cand = jnp.logical_and(b >= lob, b < hib)
                digit = jnp.where(cand, (b - lob) >> sh, 0)
                plsc.addupdate_scatter(h_v, [digit + lane_off], ones,
                                       mask=cand)
                return c

            lax.fori_loop(0, _NV, build, jnp.int32(0), unroll=8)

            accs = []
            for j in range(16):
                acc = jnp.zeros((_L,), jnp.int32)
                for sub in range(16):
                    sl = pl.ds(sub * 256 + j * _L, _L)
                    acc = acc + h_v[sl]
                    h_v[sl] = jnp.zeros((_L,), jnp.int32)
                accs.append(acc)

            carry = jnp.int32(0)
            d = jnp.int32(0)
            gtc = jnp.int32(0)
            for j in reversed(range(16)):
                acc = accs[j]
                tot = jnp.sum(acc)
                cum = jnp.cumsum(acc)
                suffix = carry + (tot - cum) + acc
                qual = suffix >= Kp
                m = jnp.sum(qual.astype(jnp.int32))
                found = jnp.logical_and(m > 0, carry < Kp)
                gt_here = carry + jnp.sum(
                    jnp.where(iota16 >= m, acc, 0))
                d = jnp.where(found, j * _L + m - 1, d)
                gtc = jnp.where(found, gt_here, gtc)
                carry = carry + tot
            lob = lob + (d << sh)
            hib = lob + (jnp.int32(1) << sh)
            Kp = Kp - gtc

        tb = lob
        need = Kp

        def tp(j, after):
            r = _NV - 1 - j
            b = ebits(r)
            e = w_v[pl.ds(r * _L, _L)]
            tie = (b == tb).astype(jnp.int32)
            csum = jnp.cumsum(tie)
            tot = jnp.sum(tie)
            after_elem = after + (tot - csum)
            keep = jnp.logical_or(
                b > tb, jnp.logical_and(tie == 1, after_elem < need))
            m_v[pl.ds(r * _L, _L)] = jnp.where(keep, e / s, 0.0)
            return after + tot

        lax.fori_loop(0, _NV, tp, jnp.int32(0), unroll=4)
        pltpu.sync_copy(m_v, out_hbm)


_sc_mask = functools.partial(
    pl.kernel,
    mesh=plsc.VectorSubcoreMesh(core_axis_name="c", subcore_axis_name="s"),
    out_type=jax.ShapeDtypeStruct((_N,), jnp.float32),
    scratch_types=[
        pltpu.VMEM((_N,), jnp.float32),
        pltpu.VMEM((_N,), jnp.float32),
        pltpu.VMEM((_N,), jnp.int32),
    ],
    compiler_params=pltpu.CompilerParams(needs_layout_passes=False),
)(_sc_mask_body)


_BLK = 512


def _stream_body(x_ref, m_ref, b_ref, o_ref):
    o_ref[...] = x_ref[...] * m_ref[...] + b_ref[...]


def _tc_stream(x, mask, bias):
    nt = x.shape[0]
    return pl.pallas_call(
        _stream_body,
        grid=(nt // _BLK,),
        in_specs=[
            pl.BlockSpec((_BLK, _N), lambda i: (i, 0)),
            pl.BlockSpec((1, _N), lambda i: (0, 0)),
            pl.BlockSpec((1, _N), lambda i: (0, 0)),
        ],
        out_specs=pl.BlockSpec((_BLK, _N), lambda i: (i, 0)),
        out_shape=jax.ShapeDtypeStruct((nt, _N), jnp.float32),
        compiler_params=pltpu.CompilerParams(
            dimension_semantics=("parallel",)),
    )(x, mask, bias)


def kernel(x, alpha, bias):
    mask = _sc_mask(alpha.reshape(_N))
    return _tc_stream(x, mask.reshape(1, _N), bias)


# speedup vs baseline: 2.2747x; 1.2208x over previous
import functools

import jax
import jax.numpy as jnp
from jax import lax
from jax.experimental import pallas as pl
from jax.experimental.pallas import tpu as pltpu
from jax.experimental.pallas import tpu_sc as plsc

_N = 4096
_K = 1024
_T = 4.0
_L = 16
_NV = _N // _L


def _sc_mask_body(alpha_hbm, out_hbm, w_v, m_v, h_v):
    cid = lax.axis_index("c")
    sid = lax.axis_index("s")

    @pl.when(jnp.logical_and(cid == 0, sid == 0))
    def _():
        pltpu.sync_copy(alpha_hbm, w_v)
        iota16 = jnp.arange(_L, dtype=jnp.int32)
        lane_off = iota16 * 256
        ones = jnp.full((_L,), 1, jnp.int32)

        def z0(i, c):
            h_v[pl.ds(i * _L, _L)] = jnp.zeros((_L,), jnp.int32)
            return c

        lax.fori_loop(0, _NV, z0, jnp.int32(0), unroll=8)

        def p1(i, mm):
            mxv, mnv = mm
            u = w_v[pl.ds(i * _L, _L)] * (1.0 / _T)
            w_v[pl.ds(i * _L, _L)] = u
            return jnp.maximum(mxv, u), jnp.minimum(mnv, u)

        mxv, mnv = lax.fori_loop(
            0, _NV, p1,
            (jnp.full((_L,), -jnp.inf, jnp.float32),
             jnp.full((_L,), jnp.inf, jnp.float32)), unroll=8)
        mx = jnp.max(mxv)
        mn = jnp.min(mnv)

        def p2(i, sv):
            e = jnp.exp(w_v[pl.ds(i * _L, _L)] - mx)
            w_v[pl.ds(i * _L, _L)] = e
            return sv + e

        sv = lax.fori_loop(0, _NV, p2, jnp.zeros((_L,), jnp.float32),
                           unroll=8)
        s = jnp.sum(sv)

        lob = jnp.max(plsc.bitcast(jnp.exp(jnp.full((_L,), mn) - mx),
                                   jnp.int32))
        hib = jnp.max(plsc.bitcast(jnp.exp(jnp.zeros((_L,), jnp.float32)),
                                   jnp.int32)) + 1

        def ebits(i):
            return plsc.bitcast(w_v[pl.ds(i * _L, _L)], jnp.int32)

        def log2_shift(rng):
            rf = (jnp.zeros((_L,), jnp.int32) + rng).astype(jnp.float32)
            rexp = ((plsc.bitcast(rf, jnp.int32) >> 23) & 255) - 127
            return jnp.max(jnp.maximum(rexp - 7, 0))

        Kp = jnp.int32(_K)
        for _lvl in range(5):
            sh = log2_shift(hib - lob - 1)

            def build(i, c, sh=sh, lob=lob, hib=hib):
                b = ebits(i)
                cand = jnp.logical_and(b >= lob, b < hib)
                digit = jnp.where(cand, (b - lob) >> sh, 0)
                plsc.addupdate_scatter(h_v, [digit + lane_off], ones,
                                       mask=cand)
                return c

            lax.fori_loop(0, _NV, build, jnp.int32(0), unroll=8)

            accs = []
            for j in range(16):
                acc = jnp.zeros((_L,), jnp.int32)
                for sub in range(16):
                    sl = pl.ds(sub * 256 + j * _L, _L)
                    acc = acc + h_v[sl]
                    h_v[sl] = jnp.zeros((_L,), jnp.int32)
                accs.append(acc)

            carry = jnp.int32(0)
            d = jnp.int32(0)
            gtc = jnp.int32(0)
            for j in reversed(range(16)):
                acc = accs[j]
                tot = jnp.sum(acc)
                cum = jnp.cumsum(acc)
                suffix = carry + (tot - cum) + acc
                qual = suffix >= Kp
                m = jnp.sum(qual.astype(jnp.int32))
                found = jnp.logical_and(m > 0, carry < Kp)
                gt_here = carry + jnp.sum(
                    jnp.where(iota16 >= m, acc, 0))
                d = jnp.where(found, j * _L + m - 1, d)
                gtc = jnp.where(found, gt_here, gtc)
                carry = carry + tot
            lob = lob + (d << sh)
            hib = lob + (jnp.int32(1) << sh)
            Kp = Kp - gtc

        tb = lob
        need = Kp

        def tp(j, after):
            r = _NV - 1 - j
            b = ebits(r)
            e = w_v[pl.ds(r * _L, _L)]
            tie = (b == tb).astype(jnp.int32)
            csum = jnp.cumsum(tie)
            tot = jnp.sum(tie)
            after_elem = after + (tot - csum)
            keep = jnp.logical_or(
                b > tb, jnp.logical_and(tie == 1, after_elem < need))
            m_v[pl.ds(r * _L, _L)] = jnp.where(keep, e / s, 0.0)
            return after + tot

        lax.fori_loop(0, _NV, tp, jnp.int32(0), unroll=4)
        pltpu.sync_copy(m_v, out_hbm)


_sc_mask = functools.partial(
    pl.kernel,
    mesh=plsc.VectorSubcoreMesh(core_axis_name="c", subcore_axis_name="s"),
    out_type=jax.ShapeDtypeStruct((_N,), jnp.float32),
    scratch_types=[
        pltpu.VMEM((_N,), jnp.float32),
        pltpu.VMEM((_N,), jnp.float32),
        pltpu.VMEM((_N,), jnp.int32),
    ],
    compiler_params=pltpu.CompilerParams(needs_layout_passes=False),
)(_sc_mask_body)


_BLK = 512


def _stream_body(x_ref, m_ref, b_ref, o_ref):
    o_ref[...] = x_ref[...] * m_ref[...] + b_ref[...]


def _tc_stream(x, mask, bias):
    nt = x.shape[0]
    return pl.pallas_call(
        _stream_body,
        grid=(nt // _BLK,),
        in_specs=[
            pl.BlockSpec((_BLK, _N), lambda i: (i, 0)),
            pl.BlockSpec((1, _N), lambda i: (0, 0)),
            pl.BlockSpec((1, _N), lambda i: (0, 0)),
        ],
        out_specs=pl.BlockSpec((_BLK, _N), lambda i: (i, 0)),
        out_shape=jax.ShapeDtypeStruct((nt, _N), jnp.float32),
        compiler_params=pltpu.CompilerParams(
            dimension_semantics=("parallel",)),
    )(x, mask, bias)


def _fused_body(alpha_ref, x_ref, bias_ref, o_ref, mask_ref):
    @pl.when(pl.program_id(0) == 0)
    def _():
        u = alpha_ref[...] * (1.0 / _T)
        mx = jnp.max(u)
        e = jnp.exp(u - mx)
        s = jnp.sum(e)
        bits = jax.lax.bitcast_convert_type(e, jnp.int32)

        def vstep(_, lohi):
            lo, hi = lohi
            mid = lo + (hi - lo) // 2
            ok = jnp.sum((bits >= mid).astype(jnp.int32)) >= _K
            return jnp.where(ok, mid, lo), jnp.where(ok, hi, mid)

        tb, _ = lax.fori_loop(0, 30, vstep,
                              (jnp.int32(0), jnp.int32(1 << 30)))
        n_ge = jnp.sum((bits >= tb).astype(jnp.int32))
        tie = bits == tb
        n_eq = jnp.sum(tie.astype(jnp.int32))
        need = _K - (n_ge - n_eq)

        idx = lax.broadcasted_iota(jnp.int32, (1, _N), 1)

        def istep(_, lohi):
            lo, hi = lohi
            mid = lo + (hi - lo) // 2
            cnt = jnp.sum(jnp.where(tie & (idx >= mid), 1, 0))
            ok = cnt >= need
            return jnp.where(ok, mid, lo), jnp.where(ok, hi, mid)

        c, _ = lax.fori_loop(0, 12, istep,
                             (jnp.int32(0), jnp.int32(_N)))
        keep = (bits > tb) | (tie & (idx >= c))
        mask_ref[...] = jnp.where(keep, e / s, 0.0)

    o_ref[...] = x_ref[...] * mask_ref[...] + bias_ref[...]


def _fused_stream(x, alpha, bias):
    nt = x.shape[0]
    return pl.pallas_call(
        _fused_body,
        grid=(nt // _BLK,),
        in_specs=[
            pl.BlockSpec((1, _N), lambda i: (0, 0)),
            pl.BlockSpec((_BLK, _N), lambda i: (i, 0)),
            pl.BlockSpec((1, _N), lambda i: (0, 0)),
        ],
        out_specs=pl.BlockSpec((_BLK, _N), lambda i: (i, 0)),
        out_shape=jax.ShapeDtypeStruct((nt, _N), jnp.float32),
        scratch_shapes=[pltpu.VMEM((1, _N), jnp.float32)],
        compiler_params=pltpu.CompilerParams(
            dimension_semantics=("arbitrary",)),
    )(alpha, x, bias)


def kernel(x, alpha, bias):
    return _fused_stream(x, alpha, bias)
